# baseline (speedup 1.0000x reference)
import functools

import jax
import jax.numpy as jnp
from jax import lax
from jax.experimental import pallas as pl
from jax.experimental.pallas import tpu as pltpu
from jax.experimental.pallas import tpu_sc as plsc

H = 768
V = 21128
B = 12
S = 128
P_CSC = 6
P_SENT = 3

NC = 2
NS = 16
NW = NC * NS
ROWS = B * S
RPW = ROWS // NW


def _sc_gather_embeds(we, ids_flat):
    mesh = plsc.VectorSubcoreMesh(
        core_axis_name="c", subcore_axis_name="s",
        num_cores=NC, num_subcores=NS)

    @functools.partial(
        pl.kernel,
        out_type=jax.ShapeDtypeStruct((ROWS, H), jnp.float32),
        mesh=mesh,
        scratch_types=[
            pltpu.VMEM((RPW,), jnp.int32),
            pltpu.VMEM((RPW, H), jnp.float32),
            pltpu.SemaphoreType.DMA,
        ],
        compiler_params=pltpu.CompilerParams(use_tc_tiling_on_sc=True),
    )
    def k(we_hbm, ids_hbm, out_hbm, idx_v, rows_v, sem):
        cid = lax.axis_index("c")
        sid = lax.axis_index("s")
        wid = sid * NC + cid
        base = wid * RPW
        pltpu.sync_copy(ids_hbm.at[pl.ds(base, RPW)], idx_v)
        pltpu.async_copy(we_hbm.at[idx_v], rows_v, sem).wait()
        pltpu.sync_copy(rows_v, out_hbm.at[pl.ds(base, RPW)])

    return k(we, ids_flat)


_BN = 512
_NB = (V + _BN - 1) // _BN
_NR = 16


def _mm_body(e_ref, sel_ref, reps_ref, we_ref, o_ref, ebuf, wbuf):
    j = pl.program_id(0)
    b = pl.program_id(1)

    @pl.when(jnp.logical_and(j == 0, b == 0))
    def _():
        sel = sel_ref[...]
        m = jnp.sum(sel.astype(jnp.float32), axis=1, keepdims=True)
        spl = lax.dot_general(
            sel, reps_ref[...], (((1,), (0,)), ((), ())),
            preferred_element_type=jnp.float32)
        e2 = e_ref[...] * (1.0 - m) + spl
        ebuf[...] = e2.astype(jnp.bfloat16)

    @pl.when(b == 0)
    def _():
        wbuf[...] = we_ref[...].astype(jnp.bfloat16)

    o_ref[0] = lax.dot_general(
        wbuf[...], ebuf[pl.ds(b * S, S), :], (((1,), (1,)), ((), ())),
        preferred_element_type=jnp.float32)


def _logits_matmul(e_g, sel, reps_pad, we):
    return pl.pallas_call(
        _mm_body,
        grid=(_NB, B),
        in_specs=[
            pl.BlockSpec((ROWS, H), lambda j, b: (0, 0)),
            pl.BlockSpec((ROWS, _NR), lambda j, b: (0, 0)),
            pl.BlockSpec((_NR, H), lambda j, b: (0, 0)),
            pl.BlockSpec((_BN, H), lambda j, b: (j, 0)),
        ],
        out_specs=pl.BlockSpec((1, _BN, S), lambda j, b: (b, j, 0)),
        out_shape=jax.ShapeDtypeStruct((B, V, S), jnp.float32),
        scratch_shapes=[pltpu.VMEM((ROWS, H), jnp.bfloat16),
                        pltpu.VMEM((_BN, H), jnp.bfloat16)],
    )(e_g, sel, reps_pad, we)


_NT = (((1,), (1,)), ((), ()))


def _run_dir(gi, whh, rev, T):
    h = jnp.zeros((1, H), jnp.float32)
    c = jnp.zeros((1, H), jnp.float32)
    rows = [None] * T
    for s in range(T):
        t = T - 1 - s if rev else s
        g = gi[t:t + 1, :] + lax.dot_general(h, whh, _NT)
        ig = jax.nn.sigmoid(g[:, :H])
        fg = jax.nn.sigmoid(g[:, H:2 * H])
        gg = jnp.tanh(g[:, 2 * H:3 * H])
        og = jax.nn.sigmoid(g[:, 3 * H:])
        c = fg * c + ig * gg
        h = og * jnp.tanh(c)
        rows[t] = h
    return jnp.concatenate(rows, axis=0)


def _layer1_body(x_ref, wf_ref, whf_ref, bf_ref, wb_ref, whb_ref, bb_ref, o_ref):
    T = x_ref.shape[0]
    x = x_ref[...]
    gif = lax.dot_general(x, wf_ref[...], _NT) + bf_ref[...]
    gib = lax.dot_general(x, wb_ref[...], _NT) + bb_ref[...]
    fw = _run_dir(gif, whf_ref[...], False, T)
    bw = _run_dir(gib, whb_ref[...], True, T)
    o_ref[...] = jnp.concatenate([fw, bw], axis=1)


def _layer2fw_body(x_ref, wf_ref, whf_ref, bf_ref, o_ref):
    T = x_ref.shape[0]
    gif = lax.dot_general(x_ref[...], wf_ref[...], _NT) + bf_ref[...]
    o_ref[...] = _run_dir(gif, whf_ref[...], False, T)


def _layer2bw_lin_body(x_ref, wb_ref, whb_ref, bb_ref, fw_ref,
                       w1_ref, b1_ref, w2_ref, b2_ref, o_ref):
    T = x_ref.shape[0]
    gib = lax.dot_general(x_ref[...], wb_ref[...], _NT) + bb_ref[...]
    bw = _run_dir(gib, whb_ref[...], True, T)
    hcat = jnp.concatenate([fw_ref[...], bw], axis=1)
    h1 = jax.nn.relu(lax.dot_general(hcat, w1_ref[...], _NT) + b1_ref[...])
    o_ref[...] = lax.dot_general(h1, w2_ref[...], _NT) + b2_ref[...]


def _call(body, out_shape, *args):
    return pl.pallas_call(
        body, out_shape=jax.ShapeDtypeStruct(out_shape, jnp.float32))(*args)


def _encode(emb, p, lin):
    T = emb.shape[0]

    def b2(v):
        return v.reshape(1, -1)

    h1 = _call(_layer1_body, (T, 2 * H),
               emb, p['Wih_l0'], p['Whh_l0'], b2(p['bih_l0'] + p['bhh_l0']),
               p['Wih_l0_r'], p['Whh_l0_r'], b2(p['bih_l0_r'] + p['bhh_l0_r']))
    fw2 = _call(_layer2fw_body, (T, H),
                h1, p['Wih_l1'], p['Whh_l1'], b2(p['bih_l1'] + p['bhh_l1']))
    rep = _call(_layer2bw_lin_body, (T, H),
                h1, p['Wih_l1_r'], p['Whh_l1_r'], b2(p['bih_l1_r'] + p['bhh_l1_r']),
                fw2, lin['W1'], b2(lin['b1']), lin['W2'], b2(lin['b2']))
    return rep


def kernel(input_ids, prompt_mask, task_id, params):
    we = params['word_emb']
    rep_csc = _encode(params['prompt_emb_csc'], params['lstm_csc'], params['lin_csc'])
    rep_tnews = _encode(params['prompt_emb_tnews'], params['lstm_tnews'], params['lin_tnews'])
    rep_afqmc = _encode(params['prompt_emb_afqmc'], params['lstm_afqmc'], params['lin_afqmc'])
    reps_pad = jnp.zeros((_NR, H), jnp.float32)
    reps_pad = lax.dynamic_update_slice(reps_pad, rep_csc, (0, 0))
    reps_pad = lax.dynamic_update_slice(reps_pad, rep_tnews, (P_CSC, 0))
    reps_pad = lax.dynamic_update_slice(reps_pad, rep_afqmc, (P_CSC + P_SENT, 0))

    pm = prompt_mask.astype(jnp.int32)
    rank = jnp.cumsum(pm, axis=1) - 1
    rep_base = jnp.where(task_id == 1, 0,
                         jnp.where(task_id == 2, P_CSC, P_CSC + P_SENT))
    col = rep_base[:, None] + rank
    sel = (col[:, :, None] == jnp.arange(_NR)[None, None, :]) & (pm[:, :, None] == 1)
    sel = sel.reshape(ROWS, _NR).astype(jnp.bfloat16)

    ids_flat = input_ids.reshape(-1).astype(jnp.int32)
    e = _sc_gather_embeds(we, ids_flat)
    logits_t = _logits_matmul(e, sel, reps_pad.astype(jnp.bfloat16), we)
    return logits_t.transpose(0, 2, 1)


# speedup vs baseline: 1.7502x; 1.7502x over previous
import functools

import jax
import jax.numpy as jnp
from jax import lax
from jax.experimental import pallas as pl
from jax.experimental.pallas import tpu as pltpu
from jax.experimental.pallas import tpu_sc as plsc

H = 768
V = 21128
B = 12
S = 128
P_CSC = 6
P_SENT = 3

NC = 2
NS = 16
NW = NC * NS
ROWS = B * S
RPW = ROWS // NW


def _sc_gather_embeds(we, ids_flat):
    mesh = plsc.VectorSubcoreMesh(
        core_axis_name="c", subcore_axis_name="s",
        num_cores=NC, num_subcores=NS)

    @functools.partial(
        pl.kernel,
        out_type=jax.ShapeDtypeStruct((ROWS, H), jnp.float32),
        mesh=mesh,
        scratch_types=[
            pltpu.VMEM((RPW,), jnp.int32),
            pltpu.VMEM((RPW, H), jnp.float32),
            pltpu.SemaphoreType.DMA,
        ],
        compiler_params=pltpu.CompilerParams(use_tc_tiling_on_sc=True),
    )
    def k(we_hbm, ids_hbm, out_hbm, idx_v, rows_v, sem):
        cid = lax.axis_index("c")
        sid = lax.axis_index("s")
        wid = sid * NC + cid
        base = wid * RPW
        pltpu.sync_copy(ids_hbm.at[pl.ds(base, RPW)], idx_v)
        pltpu.async_copy(we_hbm.at[idx_v], rows_v, sem).wait()
        pltpu.sync_copy(rows_v, out_hbm.at[pl.ds(base, RPW)])

    return k(we, ids_flat)


_BN = 512
_NB = (V + _BN - 1) // _BN
_NR = 16


def _mm_body(e_ref, sel_ref, reps_ref, we_ref, o_ref, ebuf):
    @pl.when(pl.program_id(0) == 0)
    def _():
        sel = sel_ref[...]
        m = jnp.sum(sel.astype(jnp.float32), axis=1, keepdims=True)
        spl = lax.dot_general(
            sel, reps_ref[...], (((1,), (0,)), ((), ())),
            preferred_element_type=jnp.float32)
        e2 = e_ref[...] * (1.0 - m) + spl
        ebuf[...] = e2.astype(jnp.bfloat16)

    w = we_ref[...].astype(jnp.bfloat16)
    for bb in range(B // 2):
        res = lax.dot_general(
            w, ebuf[pl.ds(bb * 2 * S, 2 * S), :], (((1,), (1,)), ((), ())),
            preferred_element_type=jnp.float32)
        o_ref[2 * bb] = res[:, :S]
        o_ref[2 * bb + 1] = res[:, S:]


def _logits_matmul(e_g, sel, reps_pad, we):
    return pl.pallas_call(
        _mm_body,
        grid=(_NB,),
        in_specs=[
            pl.BlockSpec((ROWS, H), lambda j: (0, 0)),
            pl.BlockSpec((ROWS, _NR), lambda j: (0, 0)),
            pl.BlockSpec((_NR, H), lambda j: (0, 0)),
            pl.BlockSpec((_BN, H), lambda j: (j, 0)),
        ],
        out_specs=pl.BlockSpec((B, _BN, S), lambda j: (0, j, 0)),
        out_shape=jax.ShapeDtypeStruct((B, V, S), jnp.float32),
        scratch_shapes=[pltpu.VMEM((ROWS, H), jnp.bfloat16)],
    )(e_g, sel, reps_pad, we)


_NT = (((1,), (1,)), ((), ()))


def _run_dir(gi, whh, rev, T):
    h = jnp.zeros((1, H), jnp.float32)
    c = jnp.zeros((1, H), jnp.float32)
    rows = [None] * T
    for s in range(T):
        t = T - 1 - s if rev else s
        g = gi[t:t + 1, :] + lax.dot_general(h, whh, _NT)
        ig = jax.nn.sigmoid(g[:, :H])
        fg = jax.nn.sigmoid(g[:, H:2 * H])
        gg = jnp.tanh(g[:, 2 * H:3 * H])
        og = jax.nn.sigmoid(g[:, 3 * H:])
        c = fg * c + ig * gg
        h = og * jnp.tanh(c)
        rows[t] = h
    return jnp.concatenate(rows, axis=0)


def _layer1_body(x_ref, wf_ref, whf_ref, bf_ref, wb_ref, whb_ref, bb_ref, o_ref):
    T = x_ref.shape[0]
    x = x_ref[...]
    gif = lax.dot_general(x, wf_ref[...], _NT) + bf_ref[...]
    gib = lax.dot_general(x, wb_ref[...], _NT) + bb_ref[...]
    fw = _run_dir(gif, whf_ref[...], False, T)
    bw = _run_dir(gib, whb_ref[...], True, T)
    o_ref[...] = jnp.concatenate([fw, bw], axis=1)


def _layer2fw_body(x_ref, wf_ref, whf_ref, bf_ref, o_ref):
    T = x_ref.shape[0]
    gif = lax.dot_general(x_ref[...], wf_ref[...], _NT) + bf_ref[...]
    o_ref[...] = _run_dir(gif, whf_ref[...], False, T)


def _layer2bw_lin_body(x_ref, wb_ref, whb_ref, bb_ref, fw_ref,
                       w1_ref, b1_ref, w2_ref, b2_ref, o_ref):
    T = x_ref.shape[0]
    gib = lax.dot_general(x_ref[...], wb_ref[...], _NT) + bb_ref[...]
    bw = _run_dir(gib, whb_ref[...], True, T)
    hcat = jnp.concatenate([fw_ref[...], bw], axis=1)
    h1 = jax.nn.relu(lax.dot_general(hcat, w1_ref[...], _NT) + b1_ref[...])
    o_ref[...] = lax.dot_general(h1, w2_ref[...], _NT) + b2_ref[...]


def _call(body, out_shape, *args):
    return pl.pallas_call(
        body, out_shape=jax.ShapeDtypeStruct(out_shape, jnp.float32))(*args)


def _encode(emb, p, lin):
    T = emb.shape[0]

    def b2(v):
        return v.reshape(1, -1)

    h1 = _call(_layer1_body, (T, 2 * H),
               emb, p['Wih_l0'], p['Whh_l0'], b2(p['bih_l0'] + p['bhh_l0']),
               p['Wih_l0_r'], p['Whh_l0_r'], b2(p['bih_l0_r'] + p['bhh_l0_r']))
    fw2 = _call(_layer2fw_body, (T, H),
                h1, p['Wih_l1'], p['Whh_l1'], b2(p['bih_l1'] + p['bhh_l1']))
    rep = _call(_layer2bw_lin_body, (T, H),
                h1, p['Wih_l1_r'], p['Whh_l1_r'], b2(p['bih_l1_r'] + p['bhh_l1_r']),
                fw2, lin['W1'], b2(lin['b1']), lin['W2'], b2(lin['b2']))
    return rep


def kernel(input_ids, prompt_mask, task_id, params):
    we = params['word_emb']
    rep_csc = _encode(params['prompt_emb_csc'], params['lstm_csc'], params['lin_csc'])
    rep_tnews = _encode(params['prompt_emb_tnews'], params['lstm_tnews'], params['lin_tnews'])
    rep_afqmc = _encode(params['prompt_emb_afqmc'], params['lstm_afqmc'], params['lin_afqmc'])
    reps_pad = jnp.zeros((_NR, H), jnp.float32)
    reps_pad = lax.dynamic_update_slice(reps_pad, rep_csc, (0, 0))
    reps_pad = lax.dynamic_update_slice(reps_pad, rep_tnews, (P_CSC, 0))
    reps_pad = lax.dynamic_update_slice(reps_pad, rep_afqmc, (P_CSC + P_SENT, 0))

    pm = prompt_mask.astype(jnp.int32)
    rank = jnp.cumsum(pm, axis=1) - 1
    rep_base = jnp.where(task_id == 1, 0,
                         jnp.where(task_id == 2, P_CSC, P_CSC + P_SENT))
    col = rep_base[:, None] + rank
    sel = (col[:, :, None] == jnp.arange(_NR)[None, None, :]) & (pm[:, :, None] == 1)
    sel = sel.reshape(ROWS, _NR).astype(jnp.bfloat16)

    ids_flat = input_ids.reshape(-1).astype(jnp.int32)
    e = _sc_gather_embeds(we, ids_flat)
    logits_t = _logits_matmul(e, sel, reps_pad.astype(jnp.bfloat16), we)
    return logits_t.transpose(0, 2, 1)


# speedup vs baseline: 1.7604x; 1.0058x over previous
import functools

import jax
import jax.numpy as jnp
from jax import lax
from jax.experimental import pallas as pl
from jax.experimental.pallas import tpu as pltpu
from jax.experimental.pallas import tpu_sc as plsc

H = 768
V = 21128
B = 12
S = 128
P_CSC = 6
P_SENT = 3

NC = 2
NS = 16
NW = NC * NS
ROWS = B * S
RPW = ROWS // NW


def _sc_gather_embeds(we, ids_flat):
    mesh = plsc.VectorSubcoreMesh(
        core_axis_name="c", subcore_axis_name="s",
        num_cores=NC, num_subcores=NS)

    @functools.partial(
        pl.kernel,
        out_type=jax.ShapeDtypeStruct((ROWS, H), jnp.float32),
        mesh=mesh,
        scratch_types=[
            pltpu.VMEM((RPW,), jnp.int32),
            pltpu.VMEM((RPW, H), jnp.float32),
            pltpu.SemaphoreType.DMA,
        ],
        compiler_params=pltpu.CompilerParams(use_tc_tiling_on_sc=True),
    )
    def k(we_hbm, ids_hbm, out_hbm, idx_v, rows_v, sem):
        cid = lax.axis_index("c")
        sid = lax.axis_index("s")
        wid = sid * NC + cid
        base = wid * RPW
        pltpu.sync_copy(ids_hbm.at[pl.ds(base, RPW)], idx_v)
        pltpu.async_copy(we_hbm.at[idx_v], rows_v, sem).wait()
        pltpu.sync_copy(rows_v, out_hbm.at[pl.ds(base, RPW)])

    return k(we, ids_flat)


_BN = 512
_NB = (V + _BN - 1) // _BN
_NR = 16


def _mm_body(e_ref, sel_ref, reps_ref, we_ref, o_ref, ebuf):
    @pl.when(pl.program_id(0) == 0)
    def _():
        sel = sel_ref[...]
        m = jnp.sum(sel.astype(jnp.float32), axis=1, keepdims=True)
        spl = lax.dot_general(
            sel, reps_ref[...], (((1,), (0,)), ((), ())),
            preferred_element_type=jnp.float32)
        e2 = e_ref[...] * (1.0 - m) + spl
        ebuf[...] = e2.astype(jnp.bfloat16)

    w = we_ref[...].astype(jnp.bfloat16)
    for bb in range(B // 2):
        res = lax.dot_general(
            w, ebuf[pl.ds(bb * 2 * S, 2 * S), :], (((1,), (1,)), ((), ())),
            preferred_element_type=jnp.float32)
        o_ref[2 * bb] = res[:, :S]
        o_ref[2 * bb + 1] = res[:, S:]


def _logits_matmul(e_g, sel, reps_pad, we):
    return pl.pallas_call(
        _mm_body,
        grid=(_NB,),
        in_specs=[
            pl.BlockSpec((ROWS, H), lambda j: (0, 0)),
            pl.BlockSpec((ROWS, _NR), lambda j: (0, 0)),
            pl.BlockSpec((_NR, H), lambda j: (0, 0)),
            pl.BlockSpec((_BN, H), lambda j: (j, 0)),
        ],
        out_specs=pl.BlockSpec((B, _BN, S), lambda j: (0, j, 0)),
        out_shape=jax.ShapeDtypeStruct((B, V, S), jnp.float32),
        scratch_shapes=[pltpu.VMEM((ROWS, H), jnp.bfloat16)],
    )(e_g, sel, reps_pad, we)


_NT = (((1,), (1,)), ((), ()))


def _run_dir(gi, whh, rev, T):
    h = jnp.zeros((1, H), jnp.float32)
    c = jnp.zeros((1, H), jnp.float32)
    rows = [None] * T
    for s in range(T):
        t = T - 1 - s if rev else s
        g = gi[t:t + 1, :] + lax.dot_general(h, whh, _NT)
        ig = jax.nn.sigmoid(g[:, :H])
        fg = jax.nn.sigmoid(g[:, H:2 * H])
        gg = jnp.tanh(g[:, 2 * H:3 * H])
        og = jax.nn.sigmoid(g[:, 3 * H:])
        c = fg * c + ig * gg
        h = og * jnp.tanh(c)
        rows[t] = h
    return jnp.concatenate(rows, axis=0)


def _layer1_body(x_ref, wf_ref, whf_ref, bf_ref, wb_ref, whb_ref, bb_ref, o_ref):
    T = x_ref.shape[0]
    x = x_ref[...]
    gif = lax.dot_general(x, wf_ref[...], _NT) + bf_ref[...]
    gib = lax.dot_general(x, wb_ref[...], _NT) + bb_ref[...]
    fw = _run_dir(gif, whf_ref[...], False, T)
    bw = _run_dir(gib, whb_ref[...], True, T)
    o_ref[...] = jnp.concatenate([fw, bw], axis=1)


def _layer2_body(x_ref, wf_ref, whf_ref, bf_ref, wb_ref, whb_ref, bb_ref, o_ref):
    T = x_ref.shape[0]
    x = x_ref[...]
    gif = lax.dot_general(x, wf_ref[...], _NT) + bf_ref[...]
    gib = lax.dot_general(x, wb_ref[...], _NT) + bb_ref[...]
    fw = _run_dir(gif, whf_ref[...], False, T)
    bw = _run_dir(gib, whb_ref[...], True, T)
    o_ref[...] = jnp.concatenate([fw, bw], axis=1)


def _linears3_body(hc_ref, ht_ref, ha_ref,
                   w1c, b1c, w2c, b2c,
                   w1t, b1t, w2t, b2t,
                   w1a, b1a, w2a, b2a, o_ref):
    def proj(h, w1, b1, w2, b2):
        h1 = jax.nn.relu(lax.dot_general(h, w1[...], _NT) + b1[...])
        return lax.dot_general(h1, w2[...], _NT) + b2[...]

    o_ref[...] = jnp.concatenate([
        proj(hc_ref[...], w1c, b1c, w2c, b2c),
        proj(ht_ref[...], w1t, b1t, w2t, b2t),
        proj(ha_ref[...], w1a, b1a, w2a, b2a),
        jnp.zeros((_NR - B, H), jnp.float32),
    ], axis=0)


def _call(body, out_shape, *args):
    return pl.pallas_call(
        body, out_shape=jax.ShapeDtypeStruct(out_shape, jnp.float32),
        compiler_params=pltpu.CompilerParams(
            vmem_limit_bytes=64 * 1024 * 1024))(*args)


def _b2(v):
    return v.reshape(1, -1)


def _encode(emb, p):
    T = emb.shape[0]
    h1 = _call(_layer1_body, (T, 2 * H),
               emb, p['Wih_l0'], p['Whh_l0'], _b2(p['bih_l0'] + p['bhh_l0']),
               p['Wih_l0_r'], p['Whh_l0_r'], _b2(p['bih_l0_r'] + p['bhh_l0_r']))
    h2 = _call(_layer2_body, (T, 2 * H),
               h1, p['Wih_l1'], p['Whh_l1'], _b2(p['bih_l1'] + p['bhh_l1']),
               p['Wih_l1_r'], p['Whh_l1_r'], _b2(p['bih_l1_r'] + p['bhh_l1_r']))
    return h2


def kernel(input_ids, prompt_mask, task_id, params):
    we = params['word_emb']
    h_csc = _encode(params['prompt_emb_csc'], params['lstm_csc'])
    h_tnews = _encode(params['prompt_emb_tnews'], params['lstm_tnews'])
    h_afqmc = _encode(params['prompt_emb_afqmc'], params['lstm_afqmc'])
    lc, lt, la = params['lin_csc'], params['lin_tnews'], params['lin_afqmc']
    reps_pad = _call(
        _linears3_body, (_NR, H), h_csc, h_tnews, h_afqmc,
        lc['W1'], _b2(lc['b1']), lc['W2'], _b2(lc['b2']),
        lt['W1'], _b2(lt['b1']), lt['W2'], _b2(lt['b2']),
        la['W1'], _b2(la['b1']), la['W2'], _b2(la['b2']))

    pm = prompt_mask.astype(jnp.int32)
    rank = jnp.cumsum(pm, axis=1) - 1
    rep_base = jnp.where(task_id == 1, 0,
                         jnp.where(task_id == 2, P_CSC, P_CSC + P_SENT))
    col = rep_base[:, None] + rank
    sel = (col[:, :, None] == jnp.arange(_NR)[None, None, :]) & (pm[:, :, None] == 1)
    sel = sel.reshape(ROWS, _NR).astype(jnp.bfloat16)

    ids_flat = input_ids.reshape(-1).astype(jnp.int32)
    e = _sc_gather_embeds(we, ids_flat)
    logits_t = _logits_matmul(e, sel, reps_pad.astype(jnp.bfloat16), we)
    return logits_t.transpose(0, 2, 1)


# speedup vs baseline: 1.8398x; 1.0451x over previous
import functools

import jax
import jax.numpy as jnp
from jax import lax
from jax.experimental import pallas as pl
from jax.experimental.pallas import tpu as pltpu
from jax.experimental.pallas import tpu_sc as plsc

H = 768
V = 21128
B = 12
S = 128
P_CSC = 6
P_SENT = 3

NC = 2
NS = 16
NW = NC * NS
ROWS = B * S
RPW = ROWS // NW


def _sc_gather_embeds(we, ids_flat):
    mesh = plsc.VectorSubcoreMesh(
        core_axis_name="c", subcore_axis_name="s",
        num_cores=NC, num_subcores=NS)

    @functools.partial(
        pl.kernel,
        out_type=jax.ShapeDtypeStruct((ROWS, H), jnp.float32),
        mesh=mesh,
        scratch_types=[
            pltpu.VMEM((RPW,), jnp.int32),
            pltpu.VMEM((RPW, H), jnp.float32),
            pltpu.SemaphoreType.DMA,
        ],
        compiler_params=pltpu.CompilerParams(use_tc_tiling_on_sc=True),
    )
    def k(we_hbm, ids_hbm, out_hbm, idx_v, rows_v, sem):
        cid = lax.axis_index("c")
        sid = lax.axis_index("s")
        wid = sid * NC + cid
        base = wid * RPW
        pltpu.sync_copy(ids_hbm.at[pl.ds(base, RPW)], idx_v)
        pltpu.async_copy(we_hbm.at[idx_v], rows_v, sem).wait()
        pltpu.sync_copy(rows_v, out_hbm.at[pl.ds(base, RPW)])

    return k(we, ids_flat)


_BN = 1024
_NB = (V + _BN - 1) // _BN
_NR = 16


def _mm_body(e_ref, sel_ref, reps_ref, we_ref, o_ref, ebuf):
    @pl.when(pl.program_id(0) == 0)
    def _():
        sel = sel_ref[...]
        m = jnp.sum(sel.astype(jnp.float32), axis=1, keepdims=True)
        spl = lax.dot_general(
            sel, reps_ref[...], (((1,), (0,)), ((), ())),
            preferred_element_type=jnp.float32)
        e2 = e_ref[...] * (1.0 - m) + spl
        ebuf[...] = e2.astype(jnp.bfloat16)

    w = we_ref[...].astype(jnp.bfloat16)
    for bb in range(B // 2):
        res = lax.dot_general(
            w, ebuf[pl.ds(bb * 2 * S, 2 * S), :], (((1,), (1,)), ((), ())),
            preferred_element_type=jnp.float32)
        o_ref[2 * bb] = res[:, :S]
        o_ref[2 * bb + 1] = res[:, S:]


def _logits_matmul(e_g, sel, reps_pad, we):
    return pl.pallas_call(
        _mm_body,
        grid=(_NB,),
        in_specs=[
            pl.BlockSpec((ROWS, H), lambda j: (0, 0)),
            pl.BlockSpec((ROWS, _NR), lambda j: (0, 0)),
            pl.BlockSpec((_NR, H), lambda j: (0, 0)),
            pl.BlockSpec((_BN, H), lambda j: (j, 0)),
        ],
        out_specs=pl.BlockSpec((B, _BN, S), lambda j: (0, j, 0)),
        out_shape=jax.ShapeDtypeStruct((B, V, S), jnp.float32),
        scratch_shapes=[pltpu.VMEM((ROWS, H), jnp.bfloat16)],
    )(e_g, sel, reps_pad, we)


_NT = (((1,), (1,)), ((), ()))


def _run_dir(gi, whh, rev, T):
    h = jnp.zeros((1, H), jnp.float32)
    c = jnp.zeros((1, H), jnp.float32)
    rows = [None] * T
    for s in range(T):
        t = T - 1 - s if rev else s
        g = gi[t:t + 1, :] + lax.dot_general(h, whh, _NT)
        ig = jax.nn.sigmoid(g[:, :H])
        fg = jax.nn.sigmoid(g[:, H:2 * H])
        gg = jnp.tanh(g[:, 2 * H:3 * H])
        og = jax.nn.sigmoid(g[:, 3 * H:])
        c = fg * c + ig * gg
        h = og * jnp.tanh(c)
        rows[t] = h
    return jnp.concatenate(rows, axis=0)


def _layer1_body(x_ref, wf_ref, whf_ref, bf_ref, wb_ref, whb_ref, bb_ref, o_ref):
    T = x_ref.shape[0]
    x = x_ref[...]
    gif = lax.dot_general(x, wf_ref[...], _NT) + bf_ref[...]
    gib = lax.dot_general(x, wb_ref[...], _NT) + bb_ref[...]
    fw = _run_dir(gif, whf_ref[...], False, T)
    bw = _run_dir(gib, whb_ref[...], True, T)
    o_ref[...] = jnp.concatenate([fw, bw], axis=1)


def _layer2_body(x_ref, wf_ref, whf_ref, bf_ref, wb_ref, whb_ref, bb_ref, o_ref):
    T = x_ref.shape[0]
    x = x_ref[...]
    gif = lax.dot_general(x, wf_ref[...], _NT) + bf_ref[...]
    gib = lax.dot_general(x, wb_ref[...], _NT) + bb_ref[...]
    fw = _run_dir(gif, whf_ref[...], False, T)
    bw = _run_dir(gib, whb_ref[...], True, T)
    o_ref[...] = jnp.concatenate([fw, bw], axis=1)


def _linears3_body(hc_ref, ht_ref, ha_ref,
                   w1c, b1c, w2c, b2c,
                   w1t, b1t, w2t, b2t,
                   w1a, b1a, w2a, b2a, o_ref):
    def proj(h, w1, b1, w2, b2):
        h1 = jax.nn.relu(lax.dot_general(h, w1[...], _NT) + b1[...])
        return lax.dot_general(h1, w2[...], _NT) + b2[...]

    o_ref[...] = jnp.concatenate([
        proj(hc_ref[...], w1c, b1c, w2c, b2c),
        proj(ht_ref[...], w1t, b1t, w2t, b2t),
        proj(ha_ref[...], w1a, b1a, w2a, b2a),
        jnp.zeros((_NR - B, H), jnp.float32),
    ], axis=0)


def _call(body, out_shape, *args):
    return pl.pallas_call(
        body, out_shape=jax.ShapeDtypeStruct(out_shape, jnp.float32),
        compiler_params=pltpu.CompilerParams(
            vmem_limit_bytes=64 * 1024 * 1024))(*args)


def _b2(v):
    return v.reshape(1, -1)


def _encode(emb, p):
    T = emb.shape[0]
    h1 = _call(_layer1_body, (T, 2 * H),
               emb, p['Wih_l0'], p['Whh_l0'], _b2(p['bih_l0'] + p['bhh_l0']),
               p['Wih_l0_r'], p['Whh_l0_r'], _b2(p['bih_l0_r'] + p['bhh_l0_r']))
    h2 = _call(_layer2_body, (T, 2 * H),
               h1, p['Wih_l1'], p['Whh_l1'], _b2(p['bih_l1'] + p['bhh_l1']),
               p['Wih_l1_r'], p['Whh_l1_r'], _b2(p['bih_l1_r'] + p['bhh_l1_r']))
    return h2


def kernel(input_ids, prompt_mask, task_id, params):
    we = params['word_emb']
    h_csc = _encode(params['prompt_emb_csc'], params['lstm_csc'])
    h_tnews = _encode(params['prompt_emb_tnews'], params['lstm_tnews'])
    h_afqmc = _encode(params['prompt_emb_afqmc'], params['lstm_afqmc'])
    lc, lt, la = params['lin_csc'], params['lin_tnews'], params['lin_afqmc']
    reps_pad = _call(
        _linears3_body, (_NR, H), h_csc, h_tnews, h_afqmc,
        lc['W1'], _b2(lc['b1']), lc['W2'], _b2(lc['b2']),
        lt['W1'], _b2(lt['b1']), lt['W2'], _b2(lt['b2']),
        la['W1'], _b2(la['b1']), la['W2'], _b2(la['b2']))

    pm = prompt_mask.astype(jnp.int32)
    rank = jnp.cumsum(pm, axis=1) - 1
    rep_base = jnp.where(task_id == 1, 0,
                         jnp.where(task_id == 2, P_CSC, P_CSC + P_SENT))
    col = rep_base[:, None] + rank
    sel = (col[:, :, None] == jnp.arange(_NR)[None, None, :]) & (pm[:, :, None] == 1)
    sel = sel.reshape(ROWS, _NR).astype(jnp.bfloat16)

    ids_flat = input_ids.reshape(-1).astype(jnp.int32)
    e = _sc_gather_embeds(we, ids_flat)
    logits_t = _logits_matmul(e, sel, reps_pad.astype(jnp.bfloat16), we)
    return logits_t.transpose(0, 2, 1)


# speedup vs baseline: 1.8713x; 1.0171x over previous
import functools

import jax
import jax.numpy as jnp
from jax import lax
from jax.experimental import pallas as pl
from jax.experimental.pallas import tpu as pltpu
from jax.experimental.pallas import tpu_sc as plsc

H = 768
V = 21128
B = 12
S = 128
P_CSC = 6
P_SENT = 3

NC = 2
NS = 16
NW = NC * NS
ROWS = B * S
RPW = ROWS // NW


def _sc_gather_embeds(we, ids_flat):
    mesh = plsc.VectorSubcoreMesh(
        core_axis_name="c", subcore_axis_name="s",
        num_cores=NC, num_subcores=NS)

    @functools.partial(
        pl.kernel,
        out_type=jax.ShapeDtypeStruct((ROWS, H), jnp.float32),
        mesh=mesh,
        scratch_types=[
            pltpu.VMEM((RPW,), jnp.int32),
            pltpu.VMEM((RPW, H), jnp.float32),
            pltpu.SemaphoreType.DMA,
        ],
        compiler_params=pltpu.CompilerParams(use_tc_tiling_on_sc=True),
    )
    def k(we_hbm, ids_hbm, out_hbm, idx_v, rows_v, sem):
        cid = lax.axis_index("c")
        sid = lax.axis_index("s")
        wid = sid * NC + cid
        base = wid * RPW
        pltpu.sync_copy(ids_hbm.at[pl.ds(base, RPW)], idx_v)
        pltpu.async_copy(we_hbm.at[idx_v], rows_v, sem).wait()
        pltpu.sync_copy(rows_v, out_hbm.at[pl.ds(base, RPW)])

    return k(we, ids_flat)


_BN = 2048
_NB = (V + _BN - 1) // _BN
_NR = 16


def _mm_body(e_ref, sel_ref, reps_ref, we_ref, o_ref, ebuf):
    @pl.when(pl.program_id(0) == 0)
    def _():
        sel = sel_ref[...]
        m = jnp.sum(sel.astype(jnp.float32), axis=1, keepdims=True)
        spl = lax.dot_general(
            sel, reps_ref[...], (((1,), (0,)), ((), ())),
            preferred_element_type=jnp.float32)
        e2 = e_ref[...] * (1.0 - m) + spl
        ebuf[...] = e2.astype(jnp.bfloat16)

    w = we_ref[...].astype(jnp.bfloat16)
    for bb in range(B // 2):
        res = lax.dot_general(
            w, ebuf[pl.ds(bb * 2 * S, 2 * S), :], (((1,), (1,)), ((), ())),
            preferred_element_type=jnp.float32)
        o_ref[2 * bb] = res[:, :S]
        o_ref[2 * bb + 1] = res[:, S:]


def _logits_matmul(e_g, sel, reps_pad, we):
    return pl.pallas_call(
        _mm_body,
        grid=(_NB,),
        in_specs=[
            pl.BlockSpec((ROWS, H), lambda j: (0, 0)),
            pl.BlockSpec((ROWS, _NR), lambda j: (0, 0)),
            pl.BlockSpec((_NR, H), lambda j: (0, 0)),
            pl.BlockSpec((_BN, H), lambda j: (j, 0)),
        ],
        out_specs=pl.BlockSpec((B, _BN, S), lambda j: (0, j, 0)),
        out_shape=jax.ShapeDtypeStruct((B, V, S), jnp.float32),
        scratch_shapes=[pltpu.VMEM((ROWS, H), jnp.bfloat16)],
        compiler_params=pltpu.CompilerParams(
            vmem_limit_bytes=64 * 1024 * 1024),
    )(e_g, sel, reps_pad, we)


_NT = (((1,), (1,)), ((), ()))


def _run_dir(gi, whh, rev, T):
    h = jnp.zeros((1, H), jnp.float32)
    c = jnp.zeros((1, H), jnp.float32)
    rows = [None] * T
    for s in range(T):
        t = T - 1 - s if rev else s
        g = gi[t:t + 1, :] + lax.dot_general(h, whh, _NT)
        ig = jax.nn.sigmoid(g[:, :H])
        fg = jax.nn.sigmoid(g[:, H:2 * H])
        gg = jnp.tanh(g[:, 2 * H:3 * H])
        og = jax.nn.sigmoid(g[:, 3 * H:])
        c = fg * c + ig * gg
        h = og * jnp.tanh(c)
        rows[t] = h
    return jnp.concatenate(rows, axis=0)


def _layer1_body(x_ref, wf_ref, whf_ref, bf_ref, wb_ref, whb_ref, bb_ref, o_ref):
    T = x_ref.shape[0]
    x = x_ref[...]
    gif = lax.dot_general(x, wf_ref[...], _NT) + bf_ref[...]
    gib = lax.dot_general(x, wb_ref[...], _NT) + bb_ref[...]
    fw = _run_dir(gif, whf_ref[...], False, T)
    bw = _run_dir(gib, whb_ref[...], True, T)
    o_ref[...] = jnp.concatenate([fw, bw], axis=1)


def _layer2_body(x_ref, wf_ref, whf_ref, bf_ref, wb_ref, whb_ref, bb_ref, o_ref):
    T = x_ref.shape[0]
    x = x_ref[...]
    gif = lax.dot_general(x, wf_ref[...], _NT) + bf_ref[...]
    gib = lax.dot_general(x, wb_ref[...], _NT) + bb_ref[...]
    fw = _run_dir(gif, whf_ref[...], False, T)
    bw = _run_dir(gib, whb_ref[...], True, T)
    o_ref[...] = jnp.concatenate([fw, bw], axis=1)


def _linears3_body(hc_ref, ht_ref, ha_ref,
                   w1c, b1c, w2c, b2c,
                   w1t, b1t, w2t, b2t,
                   w1a, b1a, w2a, b2a, o_ref):
    def proj(h, w1, b1, w2, b2):
        h1 = jax.nn.relu(lax.dot_general(h, w1[...], _NT) + b1[...])
        return lax.dot_general(h1, w2[...], _NT) + b2[...]

    o_ref[...] = jnp.concatenate([
        proj(hc_ref[...], w1c, b1c, w2c, b2c),
        proj(ht_ref[...], w1t, b1t, w2t, b2t),
        proj(ha_ref[...], w1a, b1a, w2a, b2a),
        jnp.zeros((_NR - B, H), jnp.float32),
    ], axis=0)


def _call(body, out_shape, *args):
    return pl.pallas_call(
        body, out_shape=jax.ShapeDtypeStruct(out_shape, jnp.float32),
        compiler_params=pltpu.CompilerParams(
            vmem_limit_bytes=64 * 1024 * 1024))(*args)


def _b2(v):
    return v.reshape(1, -1)


def _encode(emb, p):
    T = emb.shape[0]
    h1 = _call(_layer1_body, (T, 2 * H),
               emb, p['Wih_l0'], p['Whh_l0'], _b2(p['bih_l0'] + p['bhh_l0']),
               p['Wih_l0_r'], p['Whh_l0_r'], _b2(p['bih_l0_r'] + p['bhh_l0_r']))
    h2 = _call(_layer2_body, (T, 2 * H),
               h1, p['Wih_l1'], p['Whh_l1'], _b2(p['bih_l1'] + p['bhh_l1']),
               p['Wih_l1_r'], p['Whh_l1_r'], _b2(p['bih_l1_r'] + p['bhh_l1_r']))
    return h2


def kernel(input_ids, prompt_mask, task_id, params):
    we = params['word_emb']
    h_csc = _encode(params['prompt_emb_csc'], params['lstm_csc'])
    h_tnews = _encode(params['prompt_emb_tnews'], params['lstm_tnews'])
    h_afqmc = _encode(params['prompt_emb_afqmc'], params['lstm_afqmc'])
    lc, lt, la = params['lin_csc'], params['lin_tnews'], params['lin_afqmc']
    reps_pad = _call(
        _linears3_body, (_NR, H), h_csc, h_tnews, h_afqmc,
        lc['W1'], _b2(lc['b1']), lc['W2'], _b2(lc['b2']),
        lt['W1'], _b2(lt['b1']), lt['W2'], _b2(lt['b2']),
        la['W1'], _b2(la['b1']), la['W2'], _b2(la['b2']))

    pm = prompt_mask.astype(jnp.int32)
    rank = jnp.cumsum(pm, axis=1) - 1
    rep_base = jnp.where(task_id == 1, 0,
                         jnp.where(task_id == 2, P_CSC, P_CSC + P_SENT))
    col = rep_base[:, None] + rank
    sel = (col[:, :, None] == jnp.arange(_NR)[None, None, :]) & (pm[:, :, None] == 1)
    sel = sel.reshape(ROWS, _NR).astype(jnp.bfloat16)

    ids_flat = input_ids.reshape(-1).astype(jnp.int32)
    e = _sc_gather_embeds(we, ids_flat)
    logits_t = _logits_matmul(e, sel, reps_pad.astype(jnp.bfloat16), we)
    return logits_t.transpose(0, 2, 1)
